# Initial kernel scaffold; baseline (speedup 1.0000x reference)
import jax
import jax.numpy as jnp
from jax.experimental import pallas as pl


def kernel(current_test, current_question, current_tag, current_testTag, num_0, num_1, num_2, num_3, test_emb, question_emb, tag_emb, testTag_emb, cat_W, cat_b, cat_ln_g, cat_ln_b, num_W, num_b, num_ln_g, num_ln_b, out_ln_g, out_ln_b):
    raise NotImplementedError("write your pallas kernel here")


# speedup vs baseline: 2.6359x; 2.6359x over previous
import functools

import jax
import jax.numpy as jnp
from jax import lax
from jax.experimental import pallas as pl
from jax.experimental.pallas import tpu as pltpu
from jax.experimental.pallas import tpu_sc as plsc

B, L = 1024, 200
T = B * L
HID = 128
INTD = 42
HALF = 64

NC, NS = 2, 16
NW = NC * NS
TPW = T // NW
CHUNK = 128
NCHUNK = TPW // CHUNK

LN_EPS = 1e-6


def _project_body(test_ref, q_ref, tag_ref, tt_ref, w_ref, b_ref,
                  ot_ref, oq_ref, og_ref, ott_ref):
    w = w_ref[...]
    dn = (((1,), (1,)), ((), ()))
    ot_ref[...] = lax.dot_general(test_ref[...], w[:, 0 * INTD:1 * INTD], dn,
                                  preferred_element_type=jnp.float32)
    oq_ref[...] = lax.dot_general(q_ref[...], w[:, 1 * INTD:2 * INTD], dn,
                                  preferred_element_type=jnp.float32)
    og_ref[...] = lax.dot_general(tag_ref[...], w[:, 2 * INTD:3 * INTD], dn,
                                  preferred_element_type=jnp.float32)
    ott_ref[...] = lax.dot_general(tt_ref[...], w[:, 3 * INTD:4 * INTD], dn,
                                   preferred_element_type=jnp.float32) + b_ref[...]


def _project(test_emb, question_emb, tag_emb, testTag_emb, cat_W, cat_b):
    shapes = tuple(
        jax.ShapeDtypeStruct((t.shape[0], HALF), jnp.float32)
        for t in (test_emb, question_emb, tag_emb, testTag_emb))
    return pl.pallas_call(
        _project_body,
        out_shape=shapes,
    )(test_emb, question_emb, tag_emb, testTag_emb, cat_W,
      cat_b.reshape(1, HALF))


def _gather_sum_body(it_hbm, iq_hbm, ig_hbm, itt_hbm,
                     tt_hbm, tq_hbm, tg_hbm, ttt_hbm,
                     out_hbm,
                     iv_t, iv_q, iv_g, iv_tt, b0, b1, b2, b3, sem):
    wid = lax.axis_index("s") * NC + lax.axis_index("c")
    base = wid * TPW
    pltpu.sync_copy(it_hbm.at[wid], iv_t)
    pltpu.sync_copy(iq_hbm.at[wid], iv_q)
    pltpu.sync_copy(ig_hbm.at[wid], iv_g)
    pltpu.sync_copy(itt_hbm.at[wid], iv_tt)

    def chunk_body(ci, carry):
        c0 = pltpu.async_copy(tt_hbm.at[iv_t.at[ci]], b0, sem)
        c1 = pltpu.async_copy(tq_hbm.at[iv_q.at[ci]], b1, sem)
        c2 = pltpu.async_copy(tg_hbm.at[iv_g.at[ci]], b2, sem)
        c3 = pltpu.async_copy(ttt_hbm.at[iv_tt.at[ci]], b3, sem)
        c0.wait()
        c1.wait()
        c2.wait()
        c3.wait()

        def tok_body(i, carry2):
            for j in range(HALF // 16):
                s = pl.ds(j * 16, 16)
                b0[i, s] = b0[i, s] + b1[i, s] + b2[i, s] + b3[i, s]
            return carry2

        lax.fori_loop(0, CHUNK, tok_body, 0)
        pltpu.sync_copy(b0, out_hbm.at[pl.ds(base + ci * CHUNK, CHUNK)])
        return carry

    lax.fori_loop(0, NCHUNK, chunk_body, 0)


@functools.partial(jax.jit, static_argnums=())
def _gather_sum(it, iq, ig, itt, tab_t, tab_q, tab_g, tab_tt):
    mesh = plsc.VectorSubcoreMesh(core_axis_name="c", subcore_axis_name="s")
    f = pl.kernel(
        _gather_sum_body,
        out_type=jax.ShapeDtypeStruct((T, HALF), jnp.float32),
        mesh=mesh,
        compiler_params=pltpu.CompilerParams(use_tc_tiling_on_sc=False),
        scratch_types=[
            pltpu.VMEM((NCHUNK, CHUNK), jnp.int32),
            pltpu.VMEM((NCHUNK, CHUNK), jnp.int32),
            pltpu.VMEM((NCHUNK, CHUNK), jnp.int32),
            pltpu.VMEM((NCHUNK, CHUNK), jnp.int32),
            pltpu.VMEM((CHUNK, HALF), jnp.float32),
            pltpu.VMEM((CHUNK, HALF), jnp.float32),
            pltpu.VMEM((CHUNK, HALF), jnp.float32),
            pltpu.VMEM((CHUNK, HALF), jnp.float32),
            pltpu.SemaphoreType.DMA,
        ],
    )
    return f(it, iq, ig, itt, tab_t, tab_q, tab_g, tab_tt)


TB = 2048


def _ln_rows(x, g, b):
    m = jnp.mean(x, axis=-1, keepdims=True)
    v = jnp.mean((x - m) ** 2, axis=-1, keepdims=True)
    return (x - m) * lax.rsqrt(v + LN_EPS) * g + b


def _dense_body(cat_ref, num_ref, nw_ref, nb_ref,
                cg_ref, cb_ref, ng_ref, nbias2_ref, og_ref, ob_ref, out_ref):
    cat = _ln_rows(cat_ref[...], cg_ref[...], cb_ref[...])
    dn = (((1,), (1,)), ((), ()))
    numv = lax.dot_general(num_ref[...], nw_ref[...], dn,
                           preferred_element_type=jnp.float32) + nb_ref[...]
    numv = _ln_rows(numv, ng_ref[...], nbias2_ref[...])
    emb = jnp.concatenate([cat, numv], axis=-1)
    out_ref[...] = _ln_rows(emb, og_ref[...], ob_ref[...])


def _dense(cat_pre, num_feats, num_W, num_b,
           cat_ln_g, cat_ln_b, num_ln_g, num_ln_b, out_ln_g, out_ln_b):
    grid = (T // TB,)
    vec = lambda n: pl.BlockSpec((1, n), lambda i: (0, 0))
    return pl.pallas_call(
        _dense_body,
        grid=grid,
        in_specs=[
            pl.BlockSpec((TB, HALF), lambda i: (i, 0)),
            pl.BlockSpec((TB, 4), lambda i: (i, 0)),
            pl.BlockSpec((HALF, 4), lambda i: (0, 0)),
            vec(HALF), vec(HALF), vec(HALF), vec(HALF), vec(HALF),
            vec(HID), vec(HID),
        ],
        out_specs=pl.BlockSpec((TB, HID), lambda i: (i, 0)),
        out_shape=jax.ShapeDtypeStruct((T, HID), jnp.float32),
    )(cat_pre, num_feats, num_W, num_b.reshape(1, HALF),
      cat_ln_g.reshape(1, HALF), cat_ln_b.reshape(1, HALF),
      num_ln_g.reshape(1, HALF), num_ln_b.reshape(1, HALF),
      out_ln_g.reshape(1, HID), out_ln_b.reshape(1, HID))


def kernel(current_test, current_question, current_tag, current_testTag,
           num_0, num_1, num_2, num_3,
           test_emb, question_emb, tag_emb, testTag_emb,
           cat_W, cat_b, cat_ln_g, cat_ln_b,
           num_W, num_b, num_ln_g, num_ln_b,
           out_ln_g, out_ln_b):
    tab_t, tab_q, tab_g, tab_tt = _project(
        test_emb, question_emb, tag_emb, testTag_emb, cat_W, cat_b)

    def widx(a):
        return a.reshape(NW, NCHUNK, CHUNK)

    cat_pre = _gather_sum(
        widx(current_test), widx(current_question),
        widx(current_tag), widx(current_testTag),
        tab_t, tab_q, tab_g, tab_tt)

    num_feats = jnp.concatenate(
        [num_0, num_1, num_2, num_3], axis=0).reshape(B, L, 4).reshape(T, 4)

    out = _dense(cat_pre, num_feats, num_W, num_b,
                 cat_ln_g, cat_ln_b, num_ln_g, num_ln_b, out_ln_g, out_ln_b)
    return out.reshape(B, L, HID)


# speedup vs baseline: 2.6662x; 1.0115x over previous
import functools

import jax
import jax.numpy as jnp
from jax import lax
from jax.experimental import pallas as pl
from jax.experimental.pallas import tpu as pltpu
from jax.experimental.pallas import tpu_sc as plsc

B, L = 1024, 200
T = B * L
HID = 128
INTD = 42
HALF = 64

NC, NS = 2, 16
NW = NC * NS
TPW = T // NW
CHUNK = 128
NCHUNK = TPW // CHUNK

LN_EPS = 1e-6


def _project_body(test_ref, q_ref, tag_ref, tt_ref, w_ref, b_ref,
                  ot_ref, oq_ref, og_ref, ott_ref):
    w = w_ref[...]
    dn = (((1,), (1,)), ((), ()))
    ot_ref[...] = lax.dot_general(test_ref[...], w[:, 0 * INTD:1 * INTD], dn,
                                  preferred_element_type=jnp.float32
                                  ).astype(jnp.bfloat16)
    oq_ref[...] = lax.dot_general(q_ref[...], w[:, 1 * INTD:2 * INTD], dn,
                                  preferred_element_type=jnp.float32
                                  ).astype(jnp.bfloat16)
    og_ref[...] = lax.dot_general(tag_ref[...], w[:, 2 * INTD:3 * INTD], dn,
                                  preferred_element_type=jnp.float32
                                  ).astype(jnp.bfloat16)
    ott_ref[...] = (lax.dot_general(tt_ref[...], w[:, 3 * INTD:4 * INTD], dn,
                                    preferred_element_type=jnp.float32)
                    + b_ref[...]).astype(jnp.bfloat16)


def _project(test_emb, question_emb, tag_emb, testTag_emb, cat_W, cat_b):
    shapes = tuple(
        jax.ShapeDtypeStruct((t.shape[0], HALF), jnp.bfloat16)
        for t in (test_emb, question_emb, tag_emb, testTag_emb))
    return pl.pallas_call(
        _project_body,
        out_shape=shapes,
    )(test_emb, question_emb, tag_emb, testTag_emb, cat_W,
      cat_b.reshape(1, HALF))


NHALFC = NCHUNK // 2


def _gather_sum_body(it_hbm, iq_hbm, ig_hbm, itt_hbm,
                     tt_hbm, tq_hbm, tg_hbm, ttt_hbm,
                     out_hbm,
                     iv_t, iv_q, iv_g, iv_tt,
                     a0, a1, a2, a3, oa,
                     b0, b1, b2, b3, ob,
                     sga, sgb, soa, sob):
    wid = lax.axis_index("s") * NC + lax.axis_index("c")
    base = wid * TPW
    pltpu.sync_copy(it_hbm.at[wid], iv_t)
    pltpu.sync_copy(iq_hbm.at[wid], iv_q)
    pltpu.sync_copy(ig_hbm.at[wid], iv_g)
    pltpu.sync_copy(itt_hbm.at[wid], iv_tt)

    ivs = (iv_t, iv_q, iv_g, iv_tt)
    tabs = (tt_hbm, tq_hbm, tg_hbm, ttt_hbm)

    def fire_g(ci, bufs, sem):
        for tab, iv, buf in zip(tabs, ivs, bufs):
            pltpu.async_copy(tab.at[iv.at[ci]], buf, sem)

    def drain_g(ci, bufs, sem):
        for tab, iv, buf in zip(tabs, ivs, bufs):
            pltpu.make_async_copy(tab.at[iv.at[ci]], buf, sem).wait()

    def do_sum(bufs, o):
        u0, u1, u2, u3 = bufs

        def tok_body(i2, carry2):
            for u in range(2):
                i = i2 * 2 + u
                for j in range(HALF // 32):
                    s = pl.ds(j * 32, 32)
                    o[i, s] = (u0[i, s] + u1[i, s]) + (u2[i, s] + u3[i, s])
            return carry2

        lax.fori_loop(0, CHUNK // 2, tok_body, 0)

    def fire_out(ci, o, sem):
        pltpu.async_copy(o, out_hbm.at[pl.ds(base + ci * CHUNK, CHUNK)], sem)

    def wait_out(o, sem):
        pltpu.make_async_copy(o, out_hbm.at[pl.ds(base, CHUNK)], sem).wait()

    abufs = (a0, a1, a2, a3)
    bbufs = (b0, b1, b2, b3)

    fire_g(0, abufs, sga)

    def body(g, carry):
        c0 = 2 * g
        c1 = 2 * g + 1
        fire_g(c1, bbufs, sgb)
        drain_g(c0, abufs, sga)

        @pl.when(g > 0)
        def _():
            wait_out(oa, soa)

        do_sum(abufs, oa)
        fire_out(c0, oa, soa)

        @pl.when(g < NHALFC - 1)
        def _():
            fire_g(c0 + 2, abufs, sga)

        drain_g(c1, bbufs, sgb)

        @pl.when(g > 0)
        def _():
            wait_out(ob, sob)

        do_sum(bbufs, ob)
        fire_out(c1, ob, sob)
        return carry

    lax.fori_loop(0, NHALFC, body, 0)
    wait_out(oa, soa)
    wait_out(ob, sob)


@functools.partial(jax.jit, static_argnums=())
def _gather_sum(it, iq, ig, itt, tab_t, tab_q, tab_g, tab_tt):
    mesh = plsc.VectorSubcoreMesh(core_axis_name="c", subcore_axis_name="s")
    row = pltpu.VMEM((CHUNK, HALF), jnp.bfloat16)
    f = pl.kernel(
        _gather_sum_body,
        out_type=jax.ShapeDtypeStruct((T, HALF), jnp.bfloat16),
        mesh=mesh,
        compiler_params=pltpu.CompilerParams(use_tc_tiling_on_sc=False),
        scratch_types=[
            pltpu.VMEM((NCHUNK, CHUNK), jnp.int32),
            pltpu.VMEM((NCHUNK, CHUNK), jnp.int32),
            pltpu.VMEM((NCHUNK, CHUNK), jnp.int32),
            pltpu.VMEM((NCHUNK, CHUNK), jnp.int32),
            row, row, row, row, row,
            row, row, row, row, row,
            pltpu.SemaphoreType.DMA,
            pltpu.SemaphoreType.DMA,
            pltpu.SemaphoreType.DMA,
            pltpu.SemaphoreType.DMA,
        ],
    )
    return f(it, iq, ig, itt, tab_t, tab_q, tab_g, tab_tt)


TB = 2048


def _ln_rows(x, g, b):
    m = jnp.mean(x, axis=-1, keepdims=True)
    v = jnp.mean((x - m) ** 2, axis=-1, keepdims=True)
    return (x - m) * lax.rsqrt(v + LN_EPS) * g + b


def _dense_body(cat_ref, num_ref, nw_ref, nb_ref,
                cg_ref, cb_ref, ng_ref, nbias2_ref, og_ref, ob_ref, out_ref):
    cat = _ln_rows(cat_ref[...].astype(jnp.float32), cg_ref[...], cb_ref[...])
    dn = (((1,), (1,)), ((), ()))
    numv = lax.dot_general(num_ref[...], nw_ref[...], dn,
                           preferred_element_type=jnp.float32) + nb_ref[...]
    numv = _ln_rows(numv, ng_ref[...], nbias2_ref[...])
    emb = jnp.concatenate([cat, numv], axis=-1)
    out_ref[...] = _ln_rows(emb, og_ref[...], ob_ref[...])


def _dense(cat_pre, num_feats, num_W, num_b,
           cat_ln_g, cat_ln_b, num_ln_g, num_ln_b, out_ln_g, out_ln_b):
    grid = (T // TB,)
    vec = lambda n: pl.BlockSpec((1, n), lambda i: (0, 0))
    return pl.pallas_call(
        _dense_body,
        grid=grid,
        in_specs=[
            pl.BlockSpec((TB, HALF), lambda i: (i, 0)),
            pl.BlockSpec((TB, 4), lambda i: (i, 0)),
            pl.BlockSpec((HALF, 4), lambda i: (0, 0)),
            vec(HALF), vec(HALF), vec(HALF), vec(HALF), vec(HALF),
            vec(HID), vec(HID),
        ],
        out_specs=pl.BlockSpec((TB, HID), lambda i: (i, 0)),
        out_shape=jax.ShapeDtypeStruct((T, HID), jnp.float32),
    )(cat_pre, num_feats, num_W, num_b.reshape(1, HALF),
      cat_ln_g.reshape(1, HALF), cat_ln_b.reshape(1, HALF),
      num_ln_g.reshape(1, HALF), num_ln_b.reshape(1, HALF),
      out_ln_g.reshape(1, HID), out_ln_b.reshape(1, HID))


def kernel(current_test, current_question, current_tag, current_testTag,
           num_0, num_1, num_2, num_3,
           test_emb, question_emb, tag_emb, testTag_emb,
           cat_W, cat_b, cat_ln_g, cat_ln_b,
           num_W, num_b, num_ln_g, num_ln_b,
           out_ln_g, out_ln_b):
    tab_t, tab_q, tab_g, tab_tt = _project(
        test_emb, question_emb, tag_emb, testTag_emb, cat_W, cat_b)

    def widx(a):
        return a.reshape(NW, NCHUNK, CHUNK)

    cat_pre = _gather_sum(
        widx(current_test), widx(current_question),
        widx(current_tag), widx(current_testTag),
        tab_t, tab_q, tab_g, tab_tt)

    num_feats = jnp.concatenate(
        [num_0, num_1, num_2, num_3], axis=0).reshape(B, L, 4).reshape(T, 4)

    out = _dense(cat_pre, num_feats, num_W, num_b,
                 cat_ln_g, cat_ln_b, num_ln_g, num_ln_b, out_ln_g, out_ln_b)
    return out.reshape(B, L, HID)


# speedup vs baseline: 7.1175x; 2.6695x over previous
import functools

import jax
import jax.numpy as jnp
from jax import lax
from jax.experimental import pallas as pl
from jax.experimental.pallas import tpu as pltpu
from jax.experimental.pallas import tpu_sc as plsc

B, L = 1024, 200
T = B * L
HID = 128
INTD = 42
HALF = 64

NC, NS = 2, 16
NW = NC * NS
TPW = T // NW
CHUNK = 128
NCHUNK = TPW // CHUNK

LN_EPS = 1e-6


def _project_body(test_ref, q_ref, tag_ref, tt_ref, w_ref, b_ref,
                  ot_ref, oq_ref, og_ref, ott_ref):
    w = w_ref[...]
    dn = (((1,), (1,)), ((), ()))
    ot_ref[...] = lax.dot_general(test_ref[...], w[:, 0 * INTD:1 * INTD], dn,
                                  preferred_element_type=jnp.float32
                                  ).astype(jnp.bfloat16)
    oq_ref[...] = lax.dot_general(q_ref[...], w[:, 1 * INTD:2 * INTD], dn,
                                  preferred_element_type=jnp.float32
                                  ).astype(jnp.bfloat16)
    og_ref[...] = lax.dot_general(tag_ref[...], w[:, 2 * INTD:3 * INTD], dn,
                                  preferred_element_type=jnp.float32
                                  ).astype(jnp.bfloat16)
    ott_ref[...] = (lax.dot_general(tt_ref[...], w[:, 3 * INTD:4 * INTD], dn,
                                    preferred_element_type=jnp.float32)
                    + b_ref[...]).astype(jnp.bfloat16)


def _project(test_emb, question_emb, tag_emb, testTag_emb, cat_W, cat_b):
    shapes = tuple(
        jax.ShapeDtypeStruct((t.shape[0], HALF), jnp.bfloat16)
        for t in (test_emb, question_emb, tag_emb, testTag_emb))
    return pl.pallas_call(
        _project_body,
        out_shape=shapes,
    )(test_emb, question_emb, tag_emb, testTag_emb, cat_W,
      cat_b.reshape(1, HALF))


NHALFC = NCHUNK // 2


def _gather_sum_body(it_hbm, iq_hbm, ig_hbm, itt_hbm,
                     tt_hbm, tq_hbm, tg_hbm, ttt_hbm,
                     out_hbm,
                     iv_t, iv_q, iv_g, iv_tt,
                     sp_t, sp_q, sp_g, sp_tt,
                     a0, a1, a2, a3, oa,
                     b0, b1, b2, b3, ob,
                     sga, sgb, soa, sob):
    wid = lax.axis_index("s") * NC + lax.axis_index("c")
    base = wid * TPW
    @pl.when(lax.axis_index("s") == 0)
    def _():
        pltpu.sync_copy(tt_hbm, sp_t)
        pltpu.sync_copy(tq_hbm, sp_q)
        pltpu.sync_copy(tg_hbm, sp_g)
        pltpu.sync_copy(ttt_hbm, sp_tt)

    pltpu.sync_copy(it_hbm.at[wid], iv_t)
    pltpu.sync_copy(iq_hbm.at[wid], iv_q)
    pltpu.sync_copy(ig_hbm.at[wid], iv_g)
    pltpu.sync_copy(itt_hbm.at[wid], iv_tt)
    plsc.subcore_barrier()

    ivs = (iv_t, iv_q, iv_g, iv_tt)
    tabs = (sp_t, sp_q, sp_g, sp_tt)

    def fire_g(ci, bufs, sem):
        for tab, iv, buf in zip(tabs, ivs, bufs):
            pltpu.async_copy(tab.at[iv.at[ci]], buf, sem)

    def drain_g(ci, bufs, sem):
        for tab, iv, buf in zip(tabs, ivs, bufs):
            pltpu.make_async_copy(tab.at[iv.at[ci]], buf, sem).wait()

    def do_sum(bufs, o):
        u0, u1, u2, u3 = bufs

        def tok_body(i2, carry2):
            for u in range(2):
                i = i2 * 2 + u
                for j in range(HALF // 32):
                    s = pl.ds(j * 32, 32)
                    o[i, s] = (u0[i, s] + u1[i, s]) + (u2[i, s] + u3[i, s])
            return carry2

        lax.fori_loop(0, CHUNK // 2, tok_body, 0)

    def fire_out(ci, o, sem):
        pltpu.async_copy(o, out_hbm.at[pl.ds(base + ci * CHUNK, CHUNK)], sem)

    def wait_out(o, sem):
        pltpu.make_async_copy(o, out_hbm.at[pl.ds(base, CHUNK)], sem).wait()

    abufs = (a0, a1, a2, a3)
    bbufs = (b0, b1, b2, b3)

    fire_g(0, abufs, sga)

    def body(g, carry):
        c0 = 2 * g
        c1 = 2 * g + 1
        fire_g(c1, bbufs, sgb)
        drain_g(c0, abufs, sga)

        @pl.when(g > 0)
        def _():
            wait_out(oa, soa)

        do_sum(abufs, oa)
        fire_out(c0, oa, soa)

        @pl.when(g < NHALFC - 1)
        def _():
            fire_g(c0 + 2, abufs, sga)

        drain_g(c1, bbufs, sgb)

        @pl.when(g > 0)
        def _():
            wait_out(ob, sob)

        do_sum(bbufs, ob)
        fire_out(c1, ob, sob)
        return carry

    lax.fori_loop(0, NHALFC, body, 0)
    wait_out(oa, soa)
    wait_out(ob, sob)


@functools.partial(jax.jit, static_argnums=())
def _gather_sum(it, iq, ig, itt, tab_t, tab_q, tab_g, tab_tt):
    mesh = plsc.VectorSubcoreMesh(core_axis_name="c", subcore_axis_name="s")
    row = pltpu.VMEM((CHUNK, HALF), jnp.bfloat16)
    f = pl.kernel(
        _gather_sum_body,
        out_type=jax.ShapeDtypeStruct((T, HALF), jnp.bfloat16),
        mesh=mesh,
        compiler_params=pltpu.CompilerParams(use_tc_tiling_on_sc=False),
        scratch_types=[
            pltpu.VMEM((NCHUNK, CHUNK), jnp.int32),
            pltpu.VMEM((NCHUNK, CHUNK), jnp.int32),
            pltpu.VMEM((NCHUNK, CHUNK), jnp.int32),
            pltpu.VMEM((NCHUNK, CHUNK), jnp.int32),
            pltpu.VMEM_SHARED(tab_t.shape, jnp.bfloat16),
            pltpu.VMEM_SHARED(tab_q.shape, jnp.bfloat16),
            pltpu.VMEM_SHARED(tab_g.shape, jnp.bfloat16),
            pltpu.VMEM_SHARED(tab_tt.shape, jnp.bfloat16),
            row, row, row, row, row,
            row, row, row, row, row,
            pltpu.SemaphoreType.DMA,
            pltpu.SemaphoreType.DMA,
            pltpu.SemaphoreType.DMA,
            pltpu.SemaphoreType.DMA,
        ],
    )
    return f(it, iq, ig, itt, tab_t, tab_q, tab_g, tab_tt)


TB = 2048


def _ln_rows(x, g, b):
    m = jnp.mean(x, axis=-1, keepdims=True)
    v = jnp.mean((x - m) ** 2, axis=-1, keepdims=True)
    return (x - m) * lax.rsqrt(v + LN_EPS) * g + b


def _dense_body(cat_ref, num_ref, nw_ref, nb_ref,
                cg_ref, cb_ref, ng_ref, nbias2_ref, og_ref, ob_ref, out_ref):
    cat = _ln_rows(cat_ref[...].astype(jnp.float32), cg_ref[...], cb_ref[...])
    dn = (((1,), (1,)), ((), ()))
    numv = lax.dot_general(num_ref[...], nw_ref[...], dn,
                           preferred_element_type=jnp.float32) + nb_ref[...]
    numv = _ln_rows(numv, ng_ref[...], nbias2_ref[...])
    emb = jnp.concatenate([cat, numv], axis=-1)
    out_ref[...] = _ln_rows(emb, og_ref[...], ob_ref[...])


def _dense(cat_pre, num_feats, num_W, num_b,
           cat_ln_g, cat_ln_b, num_ln_g, num_ln_b, out_ln_g, out_ln_b):
    grid = (T // TB,)
    vec = lambda n: pl.BlockSpec((1, n), lambda i: (0, 0))
    return pl.pallas_call(
        _dense_body,
        grid=grid,
        in_specs=[
            pl.BlockSpec((TB, HALF), lambda i: (i, 0)),
            pl.BlockSpec((TB, 4), lambda i: (i, 0)),
            pl.BlockSpec((HALF, 4), lambda i: (0, 0)),
            vec(HALF), vec(HALF), vec(HALF), vec(HALF), vec(HALF),
            vec(HID), vec(HID),
        ],
        out_specs=pl.BlockSpec((TB, HID), lambda i: (i, 0)),
        out_shape=jax.ShapeDtypeStruct((T, HID), jnp.float32),
    )(cat_pre, num_feats, num_W, num_b.reshape(1, HALF),
      cat_ln_g.reshape(1, HALF), cat_ln_b.reshape(1, HALF),
      num_ln_g.reshape(1, HALF), num_ln_b.reshape(1, HALF),
      out_ln_g.reshape(1, HID), out_ln_b.reshape(1, HID))


def kernel(current_test, current_question, current_tag, current_testTag,
           num_0, num_1, num_2, num_3,
           test_emb, question_emb, tag_emb, testTag_emb,
           cat_W, cat_b, cat_ln_g, cat_ln_b,
           num_W, num_b, num_ln_g, num_ln_b,
           out_ln_g, out_ln_b):
    tab_t, tab_q, tab_g, tab_tt = _project(
        test_emb, question_emb, tag_emb, testTag_emb, cat_W, cat_b)

    def widx(a):
        return a.reshape(NW, NCHUNK, CHUNK)

    cat_pre = _gather_sum(
        widx(current_test), widx(current_question),
        widx(current_tag), widx(current_testTag),
        tab_t, tab_q, tab_g, tab_tt)

    num_feats = jnp.concatenate(
        [num_0, num_1, num_2, num_3], axis=0).reshape(B, L, 4).reshape(T, 4)

    out = _dense(cat_pre, num_feats, num_W, num_b,
                 cat_ln_g, cat_ln_b, num_ln_g, num_ln_b, out_ln_g, out_ln_b)
    return out.reshape(B, L, HID)
